# Initial kernel scaffold; baseline (speedup 1.0000x reference)
import jax
import jax.numpy as jnp
from jax.experimental import pallas as pl


def kernel(x, edge_index, edge_attr, batch, params):
    raise NotImplementedError("write your pallas kernel here")


# speedup vs baseline: 3.2828x; 3.2828x over previous
import functools
import math

import jax
import jax.numpy as jnp
from jax import lax
from jax.experimental import pallas as pl
from jax.experimental.pallas import tpu as pltpu
from jax.experimental.pallas import tpu_sc as plsc

_N = 10000
_D = 128
_E = 320000
_NC = 2
_NS = 16
_NW = _NC * _NS
_EPW = 10240
_EPAD = _EPW * _NW
_C = 128
_CHUNKS = _EPW // _C
_NPAD = 10112
_RPS = _NPAD // _NS

@functools.lru_cache(maxsize=None)
def _sc_prop_kernel():
    mesh = plsc.VectorSubcoreMesh(core_axis_name="c", subcore_axis_name="s")

    @functools.partial(
        pl.kernel,
        mesh=mesh,
        out_type=jax.ShapeDtypeStruct((_NC, _NPAD, _D), jnp.float32),
        scratch_types=[
            pltpu.VMEM((_C,), jnp.int32),
            pltpu.VMEM((_C,), jnp.int32),
            pltpu.VMEM((_C, _D), jnp.float32),
            pltpu.SemaphoreType.DMA,
            pltpu.VMEM_SHARED((_NPAD, _D), jnp.float32),
        ],
    )
    def _sc_prop(hs, srcp, dstp, zrows, out, src_v, dst_v, rows_v, sem, psum):
        c = lax.axis_index("c")
        s = lax.axis_index("s")
        wid = s * _NC + c
        pltpu.sync_copy(zrows.at[:], psum.at[pl.ds(s * _RPS, _RPS)])
        plsc.subcore_barrier()
        for i in range(_CHUNKS):
            base = wid * _EPW + i * _C
            pltpu.sync_copy(srcp.at[pl.ds(base, _C)], src_v)
            pltpu.sync_copy(dstp.at[pl.ds(base, _C)], dst_v)
            pltpu.async_copy(hs.at[src_v], rows_v, sem).wait()
            pltpu.sync_copy(rows_v, psum.at[dst_v], add=True)
        plsc.subcore_barrier()
        pltpu.sync_copy(psum.at[pl.ds(s * _RPS, _RPS)],
                        out.at[c, pl.ds(s * _RPS, _RPS)])

    return _sc_prop


def _kdis_body(degP_ref, dis_ref):
    deg = degP_ref[0, :_N, 0:1] + degP_ref[1, :_N, 0:1] + 1.0
    dis_ref[...] = lax.rsqrt(deg)


def _dis_pair(dis_ref):
    dis = dis_ref[...]
    return dis, dis * dis


def _relu(v):
    return jnp.maximum(v, 0.0)


def _mm(a, b):
    return jnp.dot(a, b, preferred_element_type=jnp.float32)


def _k0_body(x_ref, W_ref, dis_ref, hlin_ref, hs_ref):
    dis, _ = _dis_pair(dis_ref)
    hlin = _mm(x_ref[...], W_ref[...])
    hlin_ref[...] = hlin
    hs_ref[...] = dis * hlin


def _kconv_body(P_ref, hprev_ref, W_ref, b_ref, dis_ref, hlin_ref, hs_ref):
    dis, dis2 = _dis_pair(dis_ref)
    P = P_ref[0] + P_ref[1]
    hact = _relu(dis * P + dis2 * hprev_ref[...] + b_ref[...])
    hlin = _mm(hact, W_ref[...])
    hlin_ref[...] = hlin
    hs_ref[...] = dis * hlin


def _k4_body(P_ref, hprev_ref, b3_ref, l1W_ref, l1b_ref, Wk_ref, Wv_ref,
             dis_ref, xhK_ref, xhV_ref, hsK_ref, hsV_ref):
    dis, dis2 = _dis_pair(dis_ref)
    P = P_ref[0] + P_ref[1]
    h4 = _relu(dis * P + dis2 * hprev_ref[...] + b3_ref[...])
    xh = _mm(h4, l1W_ref[...]) + l1b_ref[...]
    xhK = _mm(xh, Wk_ref[...])
    xhV = _mm(xh, Wv_ref[...])
    xhK_ref[...] = xhK
    xhV_ref[...] = xhV
    hsK_ref[...] = dis * xhK
    hsV_ref[...] = dis * xhV


def _k5_body(P_ref, xh_ref, b_ref, dis_ref, out_ref):
    dis, dis2 = _dis_pair(dis_ref)
    P = P_ref[0] + P_ref[1]
    out_ref[...] = dis * P + dis2 * xh_ref[...] + b_ref[...]


def _softmax(s):
    m = jnp.max(s, axis=-1, keepdims=True)
    e = jnp.exp(s - m)
    return e / jnp.sum(e, axis=-1, keepdims=True)


def _attend(Q, K, V, Wo, bo):
    outs = []
    for i in range(8):
        q = Q[:, 16 * i:16 * (i + 1)]
        k = K[:, 16 * i:16 * (i + 1)]
        v = V[:, 16 * i:16 * (i + 1)]
        A = _softmax(lax.dot_general(
            q, k, (((1,), (1,)), ((), ())),
            preferred_element_type=jnp.float32) * (1.0 / math.sqrt(128.0)))
        outs.append(q + _mm(A, v))
    out = jnp.concatenate(outs, axis=1)
    return out + _relu(_mm(out, Wo) + bo)


def _k6_body(K1_ref, V1_ref, p1S_ref, p1qW_ref, p1qb_ref, p1oW_ref, p1ob_ref,
             p2qW_ref, p2qb_ref, p2kW_ref, p2kb_ref, p2vW_ref, p2vb_ref,
             p2oW_ref, p2ob_ref, p3S_ref, p3qW_ref, p3qb_ref, p3kW_ref,
             p3kb_ref, p3vW_ref, p3vb_ref, p3oW_ref, p3ob_ref, l2W_ref,
             l2b_ref, f0W_ref, f0b_ref, oW_ref, ob_ref, out_ref):
    K1 = K1_ref[...]
    V1 = V1_ref[...]
    Q1 = _mm(p1S_ref[...], p1qW_ref[...]) + p1qb_ref[...]
    o = _attend(Q1, K1, V1, p1oW_ref[...], p1ob_ref[...])
    Q2 = _mm(o, p2qW_ref[...]) + p2qb_ref[...]
    K2 = _mm(o, p2kW_ref[...]) + p2kb_ref[...]
    V2 = _mm(o, p2vW_ref[...]) + p2vb_ref[...]
    o = _attend(Q2, K2, V2, p2oW_ref[...], p2ob_ref[...])
    Q3 = _mm(p3S_ref[...], p3qW_ref[...]) + p3qb_ref[...]
    K3 = _mm(o, p3kW_ref[...]) + p3kb_ref[...]
    V3 = _mm(o, p3vW_ref[...]) + p3vb_ref[...]
    o = _attend(Q3, K3, V3, p3oW_ref[...], p3ob_ref[...])
    g = _mm(o[0:1, :], l2W_ref[...]) + l2b_ref[...]
    g = _relu(_mm(g, f0W_ref[...]) + f0b_ref[...])
    out_ref[...] = _mm(g, oW_ref[...]) + ob_ref[...]


def _tc(body, out_shapes, *args):
    return pl.pallas_call(
        body,
        out_shape=out_shapes,
    )(*args)


_GB = 2000
_G = _N // _GB


def _bs_rows(d=_D):
    return pl.BlockSpec((_GB, d), lambda i: (i, 0))


def _bs_P():
    return pl.BlockSpec((_NC, _GB, _D), lambda i: (0, i, 0))


def _bs_full(shape):
    return pl.BlockSpec(shape, lambda i: tuple(0 for _ in shape))


def _tc_rows(body, n_out, in_specs, *args):
    f32 = jnp.float32
    return pl.pallas_call(
        body,
        grid=(_G,),
        in_specs=in_specs,
        out_specs=[_bs_rows() for _ in range(n_out)],
        out_shape=[jax.ShapeDtypeStruct((_N, _D), f32)] * n_out,
    )(*args)


def kernel(x, edge_index, edge_attr, batch, params):
    p = params
    f32 = jnp.float32
    src = edge_index[0]
    dst = edge_index[1]
    npad_e = _EPAD - _E
    srcp = jnp.concatenate([src, jnp.zeros((npad_e,), jnp.int32)])
    dstp = jnp.concatenate([dst, jnp.full((npad_e,), _N, jnp.int32)])
    zrows = jnp.zeros((_RPS, _D), f32)
    ones_tab = jnp.ones((_N, _D), f32)

    def b2(name):
        return p[name].reshape(1, -1)

    _sc_prop = _sc_prop_kernel()
    degP = _sc_prop(ones_tab, srcp, dstp, zrows)
    dis = _tc(_kdis_body, jax.ShapeDtypeStruct((_N, 1), f32), degP)

    hlin, hs = _tc_rows(
        _k0_body, 2,
        [_bs_rows(), _bs_full((_D, _D)), _bs_rows(1)],
        x, p['conv0_W'], dis)
    for n in range(1, 4):
        P = _sc_prop(hs, srcp, dstp, zrows)
        hlin, hs = _tc_rows(
            _kconv_body, 2,
            [_bs_P(), _bs_rows(), _bs_full((_D, _D)), _bs_full((1, _D)),
             _bs_rows(1)],
            P, hlin, p['conv%d_W' % n], b2('conv%d_b' % (n - 1)), dis)
    P = _sc_prop(hs, srcp, dstp, zrows)
    xhK, xhV, hsK, hsV = _tc_rows(
        _k4_body, 4,
        [_bs_P(), _bs_rows(), _bs_full((1, _D)), _bs_full((_D, _D)),
         _bs_full((1, _D)), _bs_full((_D, _D)), _bs_full((_D, _D)),
         _bs_rows(1)],
        P, hlin, b2('conv3_b'), p['lin1_W'], b2('lin1_b'),
        p['p1_k_W'], p['p1_v_W'], dis)
    PK = _sc_prop(hsK, srcp, dstp, zrows)
    PV = _sc_prop(hsV, srcp, dstp, zrows)
    K1 = _tc_rows(_k5_body, 1,
                  [_bs_P(), _bs_rows(), _bs_full((1, _D)), _bs_rows(1)],
                  PK, xhK, b2('p1_k_b'), dis)[0]
    V1 = _tc_rows(_k5_body, 1,
                  [_bs_P(), _bs_rows(), _bs_full((1, _D)), _bs_rows(1)],
                  PV, xhV, b2('p1_v_b'), dis)[0]
    out = _tc(
        _k6_body,
        jax.ShapeDtypeStruct((1, 1), f32),
        K1, V1, p['p1_S'][0],
        p['p1_q_W'], b2('p1_q_b'), p['p1_o_W'], b2('p1_o_b'),
        p['p2_q_W'], b2('p2_q_b'), p['p2_k_W'], b2('p2_k_b'),
        p['p2_v_W'], b2('p2_v_b'), p['p2_o_W'], b2('p2_o_b'),
        p['p3_S'][0], p['p3_q_W'], b2('p3_q_b'), p['p3_k_W'], b2('p3_k_b'),
        p['p3_v_W'], b2('p3_v_b'), p['p3_o_W'], b2('p3_o_b'),
        p['lin2_W'], b2('lin2_b'), p['fc0_W'], b2('fc0_b'),
        p['out_W'], b2('out_b'))
    return out
